# Initial kernel scaffold; baseline (speedup 1.0000x reference)
import jax
import jax.numpy as jnp
from jax.experimental import pallas as pl


def kernel(x, edge_attr, edge_index, params):
    raise NotImplementedError("write your pallas kernel here")


# speedup vs baseline: 1.0293x; 1.0293x over previous
import functools

import jax
import jax.numpy as jnp
import numpy as np
from jax.experimental import pallas as pl

_BN_SCALE = 1.0 / np.sqrt(1.0 + 1e-5)


def _fold_mlp_A(p, splits):
    a0 = p["bn0"]["g"] * _BN_SCALE
    c0 = p["bn0"]["b"]
    W1 = p["l1"]["W"] * a0[None, :]
    b1 = p["l1"]["W"] @ c0 + p["l1"]["b"]
    a1 = p["bn1"]["g"] * _BN_SCALE
    c1 = p["bn1"]["b"]
    W2 = p["l2"]["W"] * a1[None, :]
    b2 = p["l2"]["W"] @ c1 + p["l2"]["b"]
    a2 = p["bn2"]["g"] * _BN_SCALE
    c2 = p["bn2"]["b"]
    W3 = p["l3"]["W"] * a2[None, :]
    b3 = p["l3"]["W"] @ c2 + p["l3"]["b"]
    a3 = p["bn3"]["g"] * _BN_SCALE
    c3 = p["bn3"]["b"]
    parts = []
    off = 0
    for w in splits:
        parts.append(W1[:, off:off + w].T)
        off += w
    return {"W1T": parts, "b1": b1, "W2T": W2.T, "b2": b2,
            "W3T": W3.T, "b3": b3, "a_out": a3, "c_out": c3}


def _fold_mlp_B(p, splits):
    W1 = p["l1"]["W"]
    b1 = p["l1"]["b"]
    a1 = p["bn1"]["g"] * _BN_SCALE
    c1 = p["bn1"]["b"]
    W2 = p["l2"]["W"] * a1[None, :]
    b2 = p["l2"]["W"] @ c1 + p["l2"]["b"]
    a2 = p["bn2"]["g"] * _BN_SCALE
    c2 = p["bn2"]["b"]
    W3 = p["l3"]["W"] * a2[None, :]
    b3 = p["l3"]["W"] @ c2 + p["l3"]["b"]
    a3 = p["bn3"]["g"] * _BN_SCALE
    c3 = p["bn3"]["b"]
    parts = []
    off = 0
    for w in splits:
        parts.append(W1[:, off:off + w].T)
        off += w
    return {"W1T": parts, "b1": b1, "W2T": W2.T, "b2": b2,
            "W3T": W3.T, "b3": b3, "a_out": a3, "c_out": c3}


def _mlp_tail(f, z1):
    h = jnp.maximum(z1, 0.0)
    h = jnp.maximum(jnp.dot(h, f["W2T"], preferred_element_type=jnp.float32)
                    + f["b2"], 0.0)
    h = jnp.maximum(jnp.dot(h, f["W3T"], preferred_element_type=jnp.float32)
                    + f["b3"], 0.0)
    return f["a_out"] * h + f["c_out"]


def _pick_block(n, want):
    for be in (want, want // 2, want // 4, want // 8, 320, 160, 80, 40, 8):
        if be <= n and n % be == 0:
            return be
    return n


def _stageA_body(xd_ref, xs_ref, wa_ref, wb_ref, b1_ref, w2_ref, b2_ref,
                 w3_ref, b3_ref, ao_ref, co_ref, msg_ref):
    xd = xd_ref[...]
    xs = xs_ref[...]
    z1 = (jnp.dot(xd, wa_ref[...], preferred_element_type=jnp.float32)
          + jnp.dot(xs, wb_ref[...], preferred_element_type=jnp.float32)
          + b1_ref[...])
    f = {"W2T": w2_ref[...], "b2": b2_ref[...], "W3T": w3_ref[...],
         "b3": b3_ref[...], "a_out": ao_ref[...], "c_out": co_ref[...]}
    msg_ref[...] = _mlp_tail(f, z1)


def _run_stageA(xd, xs, f):
    n_e = xd.shape[0]
    be = _pick_block(n_e, 2560)
    grid = (n_e // be,)
    wa = f["W1T"][0] - f["W1T"][1]
    wb = f["W1T"][1]
    edge_spec = lambda w: pl.BlockSpec((be, w), lambda i: (i, 0))
    full = lambda a: pl.BlockSpec(a.shape, lambda i: (0,) * a.ndim)
    return pl.pallas_call(
        _stageA_body,
        grid=grid,
        in_specs=[edge_spec(16), edge_spec(16), full(wa), full(wb),
                  full(f["b1"]), full(f["W2T"]), full(f["b2"]),
                  full(f["W3T"]), full(f["b3"]), full(f["a_out"]),
                  full(f["c_out"])],
        out_specs=edge_spec(64),
        out_shape=jax.ShapeDtypeStruct((n_e, 64), jnp.float32),
    )(xd, xs, wa, wb, f["b1"], f["W2T"], f["b2"], f["W3T"], f["b3"],
      f["a_out"], f["c_out"])


def _stageBC_body(ea_ref, hs_ref, hd_ref,
                  ewe_ref, ews_ref, ewd_ref, eb1_ref, ew2_ref, eb2_ref,
                  ew3_ref, eb3_ref, eao_ref, eco_ref,
                  cwa_ref, cwb_ref, cb1_ref, cw2_ref, cb2_ref,
                  cw3_ref, cb3_ref, cao_ref, cco_ref,
                  e1_ref, msg2_ref):
    ea = ea_ref[...]
    hs = hs_ref[...]
    hd = hd_ref[...]
    z1 = (jnp.dot(ea, ewe_ref[...], preferred_element_type=jnp.float32)
          + jnp.dot(hs, ews_ref[...], preferred_element_type=jnp.float32)
          + jnp.dot(hd, ewd_ref[...], preferred_element_type=jnp.float32)
          + eb1_ref[...])
    f = {"W2T": ew2_ref[...], "b2": eb2_ref[...], "W3T": ew3_ref[...],
         "b3": eb3_ref[...], "a_out": eao_ref[...], "c_out": eco_ref[...]}
    e1_ref[...] = _mlp_tail(f, z1)
    z1c = (jnp.dot(hd, cwa_ref[...], preferred_element_type=jnp.float32)
           + jnp.dot(hs, cwb_ref[...], preferred_element_type=jnp.float32)
           + cb1_ref[...])
    fc = {"W2T": cw2_ref[...], "b2": cb2_ref[...], "W3T": cw3_ref[...],
          "b3": cb3_ref[...], "a_out": cao_ref[...], "c_out": cco_ref[...]}
    msg2_ref[...] = _mlp_tail(fc, z1c)


def _run_stageBC(ea, hs, hd, fe, fc):
    n_e = ea.shape[0]
    be = _pick_block(n_e, 2560)
    grid = (n_e // be,)
    cwa = fc["W1T"][0] - fc["W1T"][1]
    cwb = fc["W1T"][1]
    edge_spec = lambda w: pl.BlockSpec((be, w), lambda i: (i, 0))
    full = lambda a: pl.BlockSpec(a.shape, lambda i: (0,) * a.ndim)
    args = (ea, hs, hd,
            fe["W1T"][0], fe["W1T"][1], fe["W1T"][2], fe["b1"],
            fe["W2T"], fe["b2"], fe["W3T"], fe["b3"], fe["a_out"], fe["c_out"],
            cwa, cwb, fc["b1"], fc["W2T"], fc["b2"], fc["W3T"], fc["b3"],
            fc["a_out"], fc["c_out"])
    in_specs = [edge_spec(ea.shape[1]), edge_spec(64), edge_spec(64)]
    in_specs += [full(a) for a in args[3:]]
    return pl.pallas_call(
        _stageBC_body,
        grid=grid,
        in_specs=in_specs,
        out_specs=[edge_spec(64), edge_spec(128)],
        out_shape=[jax.ShapeDtypeStruct((n_e, 64), jnp.float32),
                   jax.ShapeDtypeStruct((n_e, 128), jnp.float32)],
    )(*args)


def _stageDE_body(e1_ref, hs_ref, hd_ref,
                  ewe_ref, ews_ref, ewd_ref, eb1_ref, ew2_ref, eb2_ref,
                  ew3_ref, eb3_ref, eao_ref, eco_ref,
                  hw1_ref, hb1_ref, hw2_ref, hb2_ref, hw3_ref, hb3_ref,
                  hw45_ref, hb45_ref,
                  out_ref):
    e1 = e1_ref[...]
    hs = hs_ref[...]
    hd = hd_ref[...]
    z1 = (jnp.dot(e1, ewe_ref[...], preferred_element_type=jnp.float32)
          + jnp.dot(hs, ews_ref[...], preferred_element_type=jnp.float32)
          + jnp.dot(hd, ewd_ref[...], preferred_element_type=jnp.float32)
          + eb1_ref[...])
    f = {"W2T": ew2_ref[...], "b2": eb2_ref[...], "W3T": ew3_ref[...],
         "b3": eb3_ref[...], "a_out": eao_ref[...], "c_out": eco_ref[...]}
    e2 = _mlp_tail(f, z1)
    t = jnp.dot(e2, hw1_ref[...], preferred_element_type=jnp.float32) + hb1_ref[...]
    t = jnp.maximum(jnp.dot(t, hw2_ref[...], preferred_element_type=jnp.float32)
                    + hb2_ref[...], 0.0)
    t = jnp.maximum(jnp.dot(t, hw3_ref[...], preferred_element_type=jnp.float32)
                    + hb3_ref[...], 0.0)
    t = jnp.dot(t, hw45_ref[...], preferred_element_type=jnp.float32) + hb45_ref[...]
    out_ref[...] = jax.nn.sigmoid(t)


def _run_stageDE(e1, hs2, hd2, fe, ph):
    n_e = e1.shape[0]
    be = _pick_block(n_e, 2560)
    grid = (n_e // be,)
    w45 = ph["l4"]["W"].T @ ph["l5"]["W"].T
    b45 = ph["l4"]["b"] @ ph["l5"]["W"].T + ph["l5"]["b"]
    edge_spec = lambda w: pl.BlockSpec((be, w), lambda i: (i, 0))
    full = lambda a: pl.BlockSpec(a.shape, lambda i: (0,) * a.ndim)
    args = (e1, hs2, hd2,
            fe["W1T"][0], fe["W1T"][1], fe["W1T"][2], fe["b1"],
            fe["W2T"], fe["b2"], fe["W3T"], fe["b3"], fe["a_out"], fe["c_out"],
            ph["l1"]["W"].T, ph["l1"]["b"], ph["l2"]["W"].T, ph["l2"]["b"],
            ph["l3"]["W"].T, ph["l3"]["b"], w45, b45)
    in_specs = [edge_spec(64), edge_spec(128), edge_spec(128)]
    in_specs += [full(a) for a in args[3:]]
    return pl.pallas_call(
        _stageDE_body,
        grid=grid,
        in_specs=in_specs,
        out_specs=pl.BlockSpec((be, 1), lambda i: (i, 0)),
        out_shape=jax.ShapeDtypeStruct((n_e, 1), jnp.float32),
    )(*args)


def _stageN_body(h_ref, w1_ref, b1_ref, w2_ref, b2_ref, w34_ref, b34_ref,
                 out_ref):
    t = jnp.maximum(jnp.dot(h_ref[...], w1_ref[...],
                            preferred_element_type=jnp.float32) + b1_ref[...], 0.0)
    t = jnp.maximum(jnp.dot(t, w2_ref[...],
                            preferred_element_type=jnp.float32) + b2_ref[...], 0.0)
    t = jnp.dot(t, w34_ref[...], preferred_element_type=jnp.float32) + b34_ref[...]
    out_ref[...] = jax.nn.sigmoid(t)


def _run_stageN(h2, ph):
    n = h2.shape[0]
    bn = _pick_block(n, 2000)
    grid = (n // bn,)
    w34 = ph["l3"]["W"].T @ ph["l4"]["W"].T
    b34 = ph["l3"]["b"] @ ph["l4"]["W"].T + ph["l4"]["b"]
    args = (h2, ph["l1"]["W"].T, ph["l1"]["b"], ph["l2"]["W"].T, ph["l2"]["b"],
            w34, b34)
    full = lambda a: pl.BlockSpec(a.shape, lambda i: (0,) * a.ndim)
    in_specs = [pl.BlockSpec((bn, 128), lambda i: (i, 0))]
    in_specs += [full(a) for a in args[1:]]
    return pl.pallas_call(
        _stageN_body,
        grid=grid,
        in_specs=in_specs,
        out_specs=pl.BlockSpec((bn, 1), lambda i: (i, 0)),
        out_shape=jax.ShapeDtypeStruct((n, 1), jnp.float32),
    )(*args)


def _segmax(msg, dst, n_nodes):
    out = jax.ops.segment_max(msg, dst, num_segments=n_nodes)
    return jnp.where(jnp.isfinite(out), out, 0.0)


def kernel(x, edge_attr, edge_index, params):
    n_nodes = x.shape[0]
    src, dst = edge_index[0], edge_index[1]

    fA = _fold_mlp_A(params["nmm1"], [16, 16])
    fE1 = _fold_mlp_A(params["emm1"], [19, 64, 64])
    fC = _fold_mlp_B(params["nmm2"], [64, 64])
    fE2 = _fold_mlp_A(params["emm2"], [64, 128, 128])

    msg1 = _run_stageA(x[dst], x[src], fA)
    h1 = _segmax(msg1, dst, n_nodes)
    e1, msg2 = _run_stageBC(edge_attr, h1[src], h1[dst], fE1, fC)
    h2 = _segmax(msg2, dst, n_nodes)
    he = _run_stageDE(e1, h2[src], h2[dst], fE2, params["ehead"])
    hn = _run_stageN(h2, params["nhead"])
    return (hn, he)


# speedup vs baseline: 1.9168x; 1.8623x over previous
import functools

import jax
import jax.numpy as jnp
import numpy as np
from jax import lax
from jax.experimental import pallas as pl
from jax.experimental.pallas import tpu as pltpu
from jax.experimental.pallas import tpu_sc as plsc

_BN_SCALE = 1.0 / np.sqrt(1.0 + 1e-5)

_NUM_SC = 2
_NUM_SUBCORES = 16
_NW = _NUM_SC * _NUM_SUBCORES


def _sc_gather(table, idx, chunk):
    V, D = table.shape
    B = idx.shape[0]
    assert B % _NW == 0
    b_per_w = B // _NW
    assert b_per_w % chunk == 0 and chunk % 8 == 0
    nch = b_per_w // chunk
    mesh = plsc.VectorSubcoreMesh(core_axis_name="c", subcore_axis_name="s")

    @functools.partial(
        pl.kernel, mesh=mesh,
        out_type=jax.ShapeDtypeStruct((B, D), jnp.float32),
        compiler_params=pltpu.CompilerParams(use_tc_tiling_on_sc=False),
        scratch_types=[
            pltpu.VMEM((b_per_w,), jnp.int32),
            pltpu.VMEM((2 * chunk, D), jnp.float32),
            pltpu.SemaphoreType.DMA,
            pltpu.SemaphoreType.DMA,
        ],
    )
    def gk(table_h, idx_h, out_h, idx_v, rows_v, sem_g, sem_w):
        wid = lax.axis_index("s") * _NUM_SC + lax.axis_index("c")
        base = wid * b_per_w
        pltpu.sync_copy(idx_h.at[pl.ds(base, b_per_w)], idx_v)

        def g_start(j):
            half = (j % 2) * chunk
            pltpu.make_async_copy(
                table_h.at[idx_v.at[pl.ds(j * chunk, chunk)]],
                rows_v.at[pl.ds(half, chunk)], sem_g).start()

        def g_wait():
            pltpu.make_async_copy(
                table_h.at[pl.ds(0, chunk)],
                rows_v.at[pl.ds(0, chunk)], sem_g).wait()

        def w_start(j):
            half = (j % 2) * chunk
            pltpu.make_async_copy(
                rows_v.at[pl.ds(half, chunk)],
                out_h.at[pl.ds(base + j * chunk, chunk)], sem_w).start()

        def w_wait():
            pltpu.make_async_copy(
                rows_v.at[pl.ds(0, chunk)],
                out_h.at[pl.ds(0, chunk)], sem_w).wait()

        g_start(0)

        def body(j, carry):
            @pl.when(j >= 1)
            def _():
                w_wait()

            @pl.when(j + 1 < nch)
            def _():
                g_start(j + 1)

            g_wait()
            w_start(j)
            return carry

        lax.fori_loop(0, nch, body, 0)
        w_wait()

    return gk(table, idx)


def _fold_mlp_A(p, splits):
    a0 = p["bn0"]["g"] * _BN_SCALE
    c0 = p["bn0"]["b"]
    W1 = p["l1"]["W"] * a0[None, :]
    b1 = p["l1"]["W"] @ c0 + p["l1"]["b"]
    a1 = p["bn1"]["g"] * _BN_SCALE
    c1 = p["bn1"]["b"]
    W2 = p["l2"]["W"] * a1[None, :]
    b2 = p["l2"]["W"] @ c1 + p["l2"]["b"]
    a2 = p["bn2"]["g"] * _BN_SCALE
    c2 = p["bn2"]["b"]
    W3 = p["l3"]["W"] * a2[None, :]
    b3 = p["l3"]["W"] @ c2 + p["l3"]["b"]
    a3 = p["bn3"]["g"] * _BN_SCALE
    c3 = p["bn3"]["b"]
    parts = []
    off = 0
    for w in splits:
        parts.append(W1[:, off:off + w].T)
        off += w
    return {"W1T": parts, "b1": b1, "W2T": W2.T, "b2": b2,
            "W3T": W3.T, "b3": b3, "a_out": a3, "c_out": c3}


def _fold_mlp_B(p, splits):
    W1 = p["l1"]["W"]
    b1 = p["l1"]["b"]
    a1 = p["bn1"]["g"] * _BN_SCALE
    c1 = p["bn1"]["b"]
    W2 = p["l2"]["W"] * a1[None, :]
    b2 = p["l2"]["W"] @ c1 + p["l2"]["b"]
    a2 = p["bn2"]["g"] * _BN_SCALE
    c2 = p["bn2"]["b"]
    W3 = p["l3"]["W"] * a2[None, :]
    b3 = p["l3"]["W"] @ c2 + p["l3"]["b"]
    a3 = p["bn3"]["g"] * _BN_SCALE
    c3 = p["bn3"]["b"]
    parts = []
    off = 0
    for w in splits:
        parts.append(W1[:, off:off + w].T)
        off += w
    return {"W1T": parts, "b1": b1, "W2T": W2.T, "b2": b2,
            "W3T": W3.T, "b3": b3, "a_out": a3, "c_out": c3}


def _mlp_tail(f, z1):
    h = jnp.maximum(z1, 0.0)
    h = jnp.maximum(jnp.dot(h, f["W2T"], preferred_element_type=jnp.float32)
                    + f["b2"], 0.0)
    h = jnp.maximum(jnp.dot(h, f["W3T"], preferred_element_type=jnp.float32)
                    + f["b3"], 0.0)
    return f["a_out"] * h + f["c_out"]


def _pick_block(n, want):
    for be in (want, want // 2, want // 4, want // 8, 320, 160, 80, 40, 8):
        if be <= n and n % be == 0:
            return be
    return n


def _stageA_body(xd_ref, xs_ref, wa_ref, wb_ref, b1_ref, w2_ref, b2_ref,
                 w3_ref, b3_ref, ao_ref, co_ref, msg_ref):
    xd = xd_ref[...]
    xs = xs_ref[...]
    z1 = (jnp.dot(xd, wa_ref[...], preferred_element_type=jnp.float32)
          + jnp.dot(xs, wb_ref[...], preferred_element_type=jnp.float32)
          + b1_ref[...])
    f = {"W2T": w2_ref[...], "b2": b2_ref[...], "W3T": w3_ref[...],
         "b3": b3_ref[...], "a_out": ao_ref[...], "c_out": co_ref[...]}
    msg_ref[...] = _mlp_tail(f, z1)


def _run_stageA(g1, f):
    n_e = g1.shape[0] // 2
    be = _pick_block(n_e, 2560)
    nb = n_e // be
    grid = (nb,)
    wa = f["W1T"][0] - f["W1T"][1]
    wb = f["W1T"][1]
    dst_spec = pl.BlockSpec((be, 16), lambda i: (i + nb, 0))
    src_spec = pl.BlockSpec((be, 16), lambda i: (i, 0))
    full = lambda a: pl.BlockSpec(a.shape, lambda i: (0,) * a.ndim)
    return pl.pallas_call(
        _stageA_body,
        grid=grid,
        in_specs=[dst_spec, src_spec, full(wa), full(wb),
                  full(f["b1"]), full(f["W2T"]), full(f["b2"]),
                  full(f["W3T"]), full(f["b3"]), full(f["a_out"]),
                  full(f["c_out"])],
        out_specs=pl.BlockSpec((be, 64), lambda i: (i, 0)),
        out_shape=jax.ShapeDtypeStruct((n_e, 64), jnp.float32),
    )(g1, g1, wa, wb, f["b1"], f["W2T"], f["b2"], f["W3T"], f["b3"],
      f["a_out"], f["c_out"])


def _stageBC_body(ea_ref, hs_ref, hd_ref,
                  ewe_ref, ews_ref, ewd_ref, eb1_ref, ew2_ref, eb2_ref,
                  ew3_ref, eb3_ref, eao_ref, eco_ref,
                  cwa_ref, cwb_ref, cb1_ref, cw2_ref, cb2_ref,
                  cw3_ref, cb3_ref, cao_ref, cco_ref,
                  e1_ref, msg2_ref):
    ea = ea_ref[...]
    hs = hs_ref[...]
    hd = hd_ref[...]
    z1 = (jnp.dot(ea, ewe_ref[...], preferred_element_type=jnp.float32)
          + jnp.dot(hs, ews_ref[...], preferred_element_type=jnp.float32)
          + jnp.dot(hd, ewd_ref[...], preferred_element_type=jnp.float32)
          + eb1_ref[...])
    f = {"W2T": ew2_ref[...], "b2": eb2_ref[...], "W3T": ew3_ref[...],
         "b3": eb3_ref[...], "a_out": eao_ref[...], "c_out": eco_ref[...]}
    e1_ref[...] = _mlp_tail(f, z1)
    z1c = (jnp.dot(hd, cwa_ref[...], preferred_element_type=jnp.float32)
           + jnp.dot(hs, cwb_ref[...], preferred_element_type=jnp.float32)
           + cb1_ref[...])
    fc = {"W2T": cw2_ref[...], "b2": cb2_ref[...], "W3T": cw3_ref[...],
          "b3": cb3_ref[...], "a_out": cao_ref[...], "c_out": cco_ref[...]}
    msg2_ref[...] = _mlp_tail(fc, z1c)


def _run_stageBC(ea, g2, fe, fc):
    n_e = ea.shape[0]
    be = _pick_block(n_e, 2560)
    nb = n_e // be
    grid = (nb,)
    cwa = fc["W1T"][0] - fc["W1T"][1]
    cwb = fc["W1T"][1]
    edge_spec = lambda w: pl.BlockSpec((be, w), lambda i: (i, 0))
    src_spec = pl.BlockSpec((be, 64), lambda i: (i, 0))
    dst_spec = pl.BlockSpec((be, 64), lambda i: (i + nb, 0))
    full = lambda a: pl.BlockSpec(a.shape, lambda i: (0,) * a.ndim)
    args = (ea, g2, g2,
            fe["W1T"][0], fe["W1T"][1], fe["W1T"][2], fe["b1"],
            fe["W2T"], fe["b2"], fe["W3T"], fe["b3"], fe["a_out"], fe["c_out"],
            cwa, cwb, fc["b1"], fc["W2T"], fc["b2"], fc["W3T"], fc["b3"],
            fc["a_out"], fc["c_out"])
    in_specs = [edge_spec(ea.shape[1]), src_spec, dst_spec]
    in_specs += [full(a) for a in args[3:]]
    return pl.pallas_call(
        _stageBC_body,
        grid=grid,
        in_specs=in_specs,
        out_specs=[edge_spec(64), edge_spec(128)],
        out_shape=[jax.ShapeDtypeStruct((n_e, 64), jnp.float32),
                   jax.ShapeDtypeStruct((n_e, 128), jnp.float32)],
    )(*args)


def _stageDE_body(e1_ref, hs_ref, hd_ref,
                  ewe_ref, ews_ref, ewd_ref, eb1_ref, ew2_ref, eb2_ref,
                  ew3_ref, eb3_ref, eao_ref, eco_ref,
                  hw1_ref, hb1_ref, hw2_ref, hb2_ref, hw3_ref, hb3_ref,
                  hw45_ref, hb45_ref,
                  out_ref):
    e1 = e1_ref[...]
    hs = hs_ref[...]
    hd = hd_ref[...]
    z1 = (jnp.dot(e1, ewe_ref[...], preferred_element_type=jnp.float32)
          + jnp.dot(hs, ews_ref[...], preferred_element_type=jnp.float32)
          + jnp.dot(hd, ewd_ref[...], preferred_element_type=jnp.float32)
          + eb1_ref[...])
    f = {"W2T": ew2_ref[...], "b2": eb2_ref[...], "W3T": ew3_ref[...],
         "b3": eb3_ref[...], "a_out": eao_ref[...], "c_out": eco_ref[...]}
    e2 = _mlp_tail(f, z1)
    t = jnp.dot(e2, hw1_ref[...], preferred_element_type=jnp.float32) + hb1_ref[...]
    t = jnp.maximum(jnp.dot(t, hw2_ref[...], preferred_element_type=jnp.float32)
                    + hb2_ref[...], 0.0)
    t = jnp.maximum(jnp.dot(t, hw3_ref[...], preferred_element_type=jnp.float32)
                    + hb3_ref[...], 0.0)
    t = jnp.dot(t, hw45_ref[...], preferred_element_type=jnp.float32) + hb45_ref[...]
    out_ref[...] = jax.nn.sigmoid(t)


def _run_stageDE(e1, g3, fe, ph):
    n_e = e1.shape[0]
    be = _pick_block(n_e, 2560)
    nb = n_e // be
    grid = (nb,)
    w45 = ph["l4"]["W"].T @ ph["l5"]["W"].T
    b45 = ph["l4"]["b"] @ ph["l5"]["W"].T + ph["l5"]["b"]
    edge_spec = lambda w: pl.BlockSpec((be, w), lambda i: (i, 0))
    src_spec = pl.BlockSpec((be, 128), lambda i: (i, 0))
    dst_spec = pl.BlockSpec((be, 128), lambda i: (i + nb, 0))
    full = lambda a: pl.BlockSpec(a.shape, lambda i: (0,) * a.ndim)
    args = (e1, g3, g3,
            fe["W1T"][0], fe["W1T"][1], fe["W1T"][2], fe["b1"],
            fe["W2T"], fe["b2"], fe["W3T"], fe["b3"], fe["a_out"], fe["c_out"],
            ph["l1"]["W"].T, ph["l1"]["b"], ph["l2"]["W"].T, ph["l2"]["b"],
            ph["l3"]["W"].T, ph["l3"]["b"], w45, b45)
    in_specs = [edge_spec(64), src_spec, dst_spec]
    in_specs += [full(a) for a in args[3:]]
    return pl.pallas_call(
        _stageDE_body,
        grid=grid,
        in_specs=in_specs,
        out_specs=pl.BlockSpec((be, 1), lambda i: (i, 0)),
        out_shape=jax.ShapeDtypeStruct((n_e, 1), jnp.float32),
    )(*args)


def _stageN_body(h_ref, w1_ref, b1_ref, w2_ref, b2_ref, w34_ref, b34_ref,
                 out_ref):
    t = jnp.maximum(jnp.dot(h_ref[...], w1_ref[...],
                            preferred_element_type=jnp.float32) + b1_ref[...], 0.0)
    t = jnp.maximum(jnp.dot(t, w2_ref[...],
                            preferred_element_type=jnp.float32) + b2_ref[...], 0.0)
    t = jnp.dot(t, w34_ref[...], preferred_element_type=jnp.float32) + b34_ref[...]
    out_ref[...] = jax.nn.sigmoid(t)


def _run_stageN(h2, ph):
    n = h2.shape[0]
    bn = _pick_block(n, 2000)
    grid = (n // bn,)
    w34 = ph["l3"]["W"].T @ ph["l4"]["W"].T
    b34 = ph["l3"]["b"] @ ph["l4"]["W"].T + ph["l4"]["b"]
    args = (h2, ph["l1"]["W"].T, ph["l1"]["b"], ph["l2"]["W"].T, ph["l2"]["b"],
            w34, b34)
    full = lambda a: pl.BlockSpec(a.shape, lambda i: (0,) * a.ndim)
    in_specs = [pl.BlockSpec((bn, 128), lambda i: (i, 0))]
    in_specs += [full(a) for a in args[1:]]
    return pl.pallas_call(
        _stageN_body,
        grid=grid,
        in_specs=in_specs,
        out_specs=pl.BlockSpec((bn, 1), lambda i: (i, 0)),
        out_shape=jax.ShapeDtypeStruct((n, 1), jnp.float32),
    )(*args)


def _segmax(msg, dst, n_nodes):
    out = jax.ops.segment_max(msg, dst, num_segments=n_nodes)
    return jnp.where(jnp.isfinite(out), out, 0.0)


def kernel(x, edge_attr, edge_index, params):
    n_nodes = x.shape[0]
    src, dst = edge_index[0], edge_index[1]

    fA = _fold_mlp_A(params["nmm1"], [16, 16])
    fE1 = _fold_mlp_A(params["emm1"], [19, 64, 64])
    fC = _fold_mlp_B(params["nmm2"], [64, 64])
    fE2 = _fold_mlp_A(params["emm2"], [64, 128, 128])

    idx_flat = edge_index.reshape(-1)

    g1 = _sc_gather(x, idx_flat, 2000)
    msg1 = _run_stageA(g1, fA)
    h1 = _segmax(msg1, dst, n_nodes)
    g2 = _sc_gather(h1, idx_flat, 800)
    e1, msg2 = _run_stageBC(edge_attr, g2, fE1, fC)
    h2 = _segmax(msg2, dst, n_nodes)
    g3 = _sc_gather(h2, idx_flat, 400)
    he = _run_stageDE(e1, g3, fE2, params["ehead"])
    hn = _run_stageN(h2, params["nhead"])
    return (hn, he)
